# Initial kernel scaffold; baseline (speedup 1.0000x reference)
import jax
import jax.numpy as jnp
from jax.experimental import pallas as pl


def kernel(node_feat, node_pos, node_vel, edge_index, edge_attr, batch, params):
    raise NotImplementedError("write your pallas kernel here")


# speedup vs baseline: 2.7505x; 2.7505x over previous
import functools

import jax
import jax.numpy as jnp
from jax import lax
from jax.experimental import pallas as pl
from jax.experimental.pallas import tpu as pltpu
from jax.experimental.pallas import tpu_sc as plsc

N = 10000
E = 320000
HD = 64
NG = 16
NL = 4

NC = 2
NS = 16
NW = NC * NS
EPW = E // NW
CH = 80
NCHUNK = EPW // CH

def _silu(x):
    return x * jax.nn.sigmoid(x)


def _sc_gather_body(row_h, col_h, a_h, b_h, pos_h, g_h, d4_h,
                    ridx, cidx, abuf, bbuf, dbuf, posv, sem):
    wid = lax.axis_index("s") * NC + lax.axis_index("c")
    pltpu.sync_copy(pos_h, posv)
    base0 = wid * EPW

    def chunk(i, carry):
        base = base0 + i * CH
        pltpu.sync_copy(row_h.at[pl.ds(base, CH)], ridx)
        pltpu.sync_copy(col_h.at[pl.ds(base, CH)], cidx)
        cpa = pltpu.async_copy(a_h.at[ridx], abuf, sem)
        cpb = pltpu.async_copy(b_h.at[cidx], bbuf, sem)
        zero16 = jnp.zeros((16,), jnp.float32)
        for j in range(CH // 16):
            off = j * 16
            r16 = ridx[pl.ds(off, 16)]
            c16 = cidx[pl.ds(off, 16)]
            ids = (lax.iota(jnp.int32, 16) + off) * 4
            for comp in range(3):
                pr = plsc.load_gather(posv, [r16 * 4 + comp])
                pc = plsc.load_gather(posv, [c16 * 4 + comp])
                plsc.store_scatter(dbuf, [ids + comp], pr - pc)
            plsc.store_scatter(dbuf, [ids + 3], zero16)
        cpa.wait()
        cpb.wait()

        def addrow(r, c2):
            for cc in range(HD // 16):
                sl = pl.ds(cc * 16, 16)
                abuf[r, sl] = abuf[r, sl] + bbuf[r, sl]
            return c2

        lax.fori_loop(0, CH, addrow, 0, unroll=2)
        pltpu.sync_copy(abuf, g_h.at[pl.ds(base, CH)])
        pltpu.sync_copy(dbuf, d4_h.at[pl.ds(base * 4, CH * 4)])
        return carry

    lax.fori_loop(0, NCHUNK, chunk, 0)


SW = 80
_ZROWS = 624


def _sc_scatter_body(row_h, eo_h, z_h, s_h, acc, ridx, ebuf):
    cid = lax.axis_index("c")
    sid = lax.axis_index("s")
    wid = sid * NC + cid
    r0 = sid * _ZROWS
    pltpu.sync_copy(z_h.at[pl.ds(r0, _ZROWS)], acc.at[pl.ds(r0, _ZROWS)])

    @pl.when(sid == NS - 1)
    def _():
        t0 = NS * _ZROWS
        pltpu.sync_copy(z_h.at[pl.ds(t0, N - t0)], acc.at[pl.ds(t0, N - t0)])

    plsc.subcore_barrier()

    base0 = wid * EPW

    def chunk(i, carry):
        base = base0 + i * CH
        pltpu.sync_copy(row_h.at[pl.ds(base, CH)], ridx)
        pltpu.sync_copy(eo_h.at[pl.ds(base, CH)], ebuf)
        pltpu.sync_copy(ebuf, acc.at[ridx], add=True)
        return carry

    lax.fori_loop(0, NCHUNK, chunk, 0)
    plsc.subcore_barrier()
    pltpu.sync_copy(acc.at[pl.ds(r0, _ZROWS)], s_h.at[cid, pl.ds(r0, _ZROWS)])

    @pl.when(sid == NS - 1)
    def _():
        t0 = NS * _ZROWS
        pltpu.sync_copy(acc.at[pl.ds(t0, N - t0)], s_h.at[cid, pl.ds(t0, N - t0)])


@functools.cache
def _sc_kernels():
    mesh = plsc.VectorSubcoreMesh(
        core_axis_name="c", subcore_axis_name="s",
        num_cores=NC, num_subcores=NS,
    )
    params = pltpu.CompilerParams(
        needs_layout_passes=False, use_tc_tiling_on_sc=False
    )
    gather = pl.kernel(
        _sc_gather_body,
        out_type=[
            jax.ShapeDtypeStruct((E, HD), jnp.float32),
            jax.ShapeDtypeStruct((E * 4,), jnp.float32),
        ],
        mesh=mesh,
        compiler_params=params,
        scratch_types=[
            pltpu.VMEM((CH,), jnp.int32),
            pltpu.VMEM((CH,), jnp.int32),
            pltpu.VMEM((CH, HD), jnp.float32),
            pltpu.VMEM((CH, HD), jnp.float32),
            pltpu.VMEM((CH * 4,), jnp.float32),
            pltpu.VMEM((N * 4,), jnp.float32),
            pltpu.SemaphoreType.DMA,
        ],
    )
    scatter = pl.kernel(
        _sc_scatter_body,
        out_type=jax.ShapeDtypeStruct((NC, N, SW), jnp.float32),
        mesh=mesh,
        compiler_params=params,
        scratch_types=[
            pltpu.VMEM_SHARED((N, SW), jnp.float32),
            pltpu.VMEM((CH,), jnp.int32),
            pltpu.VMEM((CH, SW), jnp.float32),
        ],
    )
    return gather, scatter


KT = 2000


def _edge_body(g_ref, d4_ref, ea_ref, w1c_ref, w1d_ref, b1_ref, w2_ref,
               b2_ref, wp1_ref, bp1_ref, wp2_ref, bp2_ref, eo_ref):
    d4 = d4_ref[...]
    dist = jnp.sum(d4 * d4, axis=-1, keepdims=True)
    h = g_ref[...] + jnp.dot(ea_ref[...], w1c_ref[...],
                             preferred_element_type=jnp.float32)
    h = h + dist * w1d_ref[...] + b1_ref[...]
    h = _silu(h)
    msg = _silu(jnp.dot(h, w2_ref[...], preferred_element_type=jnp.float32)
                + b2_ref[...])
    u = _silu(jnp.dot(msg, wp1_ref[...], preferred_element_type=jnp.float32)
              + bp1_ref[...])
    w = (jnp.dot(u, wp2_ref[...], preferred_element_type=jnp.float32)
         + bp2_ref[...])[:, 0:1]
    cnt1 = (lax.broadcasted_iota(jnp.int32, (1, 4), 1) == 3).astype(jnp.float32)
    eo_ref[:, 0:HD] = msg
    eo_ref[:, HD:HD + 4] = d4 * w + cnt1
    eo_ref[:, HD + 4:SW] = jnp.zeros((d4.shape[0], SW - HD - 4), jnp.float32)


def _full(shape):
    return pl.BlockSpec(shape, lambda i: tuple(0 for _ in shape))


_edge_call = pl.pallas_call(
    _edge_body,
    grid=(E // KT,),
    in_specs=[
        pl.BlockSpec((KT, HD), lambda i: (i, 0)),
        pl.BlockSpec((KT, 4), lambda i: (i, 0)),
        pl.BlockSpec((KT, 4), lambda i: (i, 0)),
        _full((4, HD)),
        _full((1, HD)),
        _full((1, HD)),
        _full((HD, HD)),
        _full((1, HD)),
        _full((HD, HD)),
        _full((1, HD)),
        _full((HD, 8)),
        _full((1, 8)),
    ],
    out_specs=pl.BlockSpec((KT, SW), lambda i: (i, 0)),
    out_shape=jax.ShapeDtypeStruct((E, SW), jnp.float32),
)


NT = 1000


def _node_body(feat_ref, s_ref, pos_ref, vel_ref,
               wn1a_ref, wn1b_ref, bn1_ref, wn2_ref, bn2_ref,
               wv1_ref, bv1_ref, wv2_ref, bv2_ref,
               wa_ref, wb_ref,
               feat_o, a_o, b_o, pos_o):
    feat = feat_ref[...]
    s2 = s_ref[0] + s_ref[1]
    sm = s2[:, 0:HD]
    sp = s2[:, HD:HD + 4]
    cnt = jnp.maximum(sp[:, 3:4], 1.0)
    inv = 1.0 / cnt
    msg_agg = sm * inv
    pos_agg = sp * inv
    hv = _silu(jnp.dot(feat, wv1_ref[...], preferred_element_type=jnp.float32)
               + bv1_ref[...])
    s = (jnp.dot(hv, wv2_ref[...], preferred_element_type=jnp.float32)
         + bv2_ref[...])[:, 0:1]
    mask = (lax.broadcasted_iota(jnp.int32, (1, 4), 1) < 3).astype(jnp.float32)
    pos_o[...] = (pos_ref[...] + pos_agg + s * vel_ref[...]) * mask
    hn = _silu(jnp.dot(feat, wn1a_ref[...], preferred_element_type=jnp.float32)
               + jnp.dot(msg_agg, wn1b_ref[...],
                         preferred_element_type=jnp.float32)
               + bn1_ref[...])
    fnew = (jnp.dot(hn, wn2_ref[...], preferred_element_type=jnp.float32)
            + bn2_ref[...])
    feat_o[...] = fnew
    a_o[...] = jnp.dot(fnew, wa_ref[...], preferred_element_type=jnp.float32)
    b_o[...] = jnp.dot(fnew, wb_ref[...], preferred_element_type=jnp.float32)


_node_call = pl.pallas_call(
    _node_body,
    grid=(N // NT,),
    in_specs=[
        pl.BlockSpec((NT, HD), lambda i: (i, 0)),
        pl.BlockSpec((NC, NT, SW), lambda i: (0, i, 0)),
        pl.BlockSpec((NT, 4), lambda i: (i, 0)),
        pl.BlockSpec((NT, 4), lambda i: (i, 0)),
        _full((HD, HD)),
        _full((HD, HD)),
        _full((1, HD)),
        _full((HD, HD)),
        _full((1, HD)),
        _full((HD, HD)),
        _full((1, HD)),
        _full((HD, 8)),
        _full((1, 8)),
        _full((HD, HD)),
        _full((HD, HD)),
    ],
    out_specs=[
        pl.BlockSpec((NT, HD), lambda i: (i, 0)),
        pl.BlockSpec((NT, HD), lambda i: (i, 0)),
        pl.BlockSpec((NT, HD), lambda i: (i, 0)),
        pl.BlockSpec((NT, 4), lambda i: (i, 0)),
    ],
    out_shape=[
        jax.ShapeDtypeStruct((N, HD), jnp.float32),
        jax.ShapeDtypeStruct((N, HD), jnp.float32),
        jax.ShapeDtypeStruct((N, HD), jnp.float32),
        jax.ShapeDtypeStruct((N, 4), jnp.float32),
    ],
)


def _prologue_body(nf_ref, pos_ref, brow_ref, bcol_ref,
                   we_ref, be_ref, wc1_ref, bc1_ref, wc2_ref, bc2_ref,
                   wa_ref, wb_ref, feat_o, a_o, b_o):
    pos4 = pos_ref[...]
    onehot = (lax.broadcasted_iota(jnp.int32, (NG, 1), 0)
              == brow_ref[...]).astype(jnp.float32)
    cnt = jnp.maximum(jnp.sum(onehot, axis=1, keepdims=True), 1.0)
    center = jnp.dot(onehot, pos4, preferred_element_type=jnp.float32) / cnt
    onehot_t = (bcol_ref[...]
                == lax.broadcasted_iota(jnp.int32, (1, NG), 1)).astype(
                    jnp.float32)
    posc = pos4 - jnp.dot(onehot_t, center, preferred_element_type=jnp.float32)
    scalar = jnp.sqrt(jnp.sum(posc * posc, axis=-1, keepdims=True))
    hc = _silu(scalar * wc1_ref[...] + bc1_ref[...])
    feat = (jnp.dot(nf_ref[...], we_ref[...],
                    preferred_element_type=jnp.float32) + be_ref[...]
            + jnp.dot(hc, wc2_ref[...], preferred_element_type=jnp.float32)
            + bc2_ref[...])
    feat_o[...] = feat
    a_o[...] = jnp.dot(feat, wa_ref[...], preferred_element_type=jnp.float32)
    b_o[...] = jnp.dot(feat, wb_ref[...], preferred_element_type=jnp.float32)


_prologue_call = pl.pallas_call(
    _prologue_body,
    grid=(1,),
    in_specs=[
        _full((N, 2)),
        _full((N, 4)),
        _full((1, N)),
        _full((N, 1)),
        _full((2, HD)),
        _full((1, HD)),
        _full((1, HD)),
        _full((1, HD)),
        _full((HD, HD)),
        _full((1, HD)),
        _full((HD, HD)),
        _full((HD, HD)),
    ],
    out_specs=[
        _full((N, HD)),
        _full((N, HD)),
        _full((N, HD)),
    ],
    out_shape=[
        jax.ShapeDtypeStruct((N, HD), jnp.float32),
        jax.ShapeDtypeStruct((N, HD), jnp.float32),
        jax.ShapeDtypeStruct((N, HD), jnp.float32),
    ],
)


def _split_w1(lp):
    w1 = lp["msg"]["W1"]
    return (w1[:HD], w1[HD:2 * HD], w1[2 * HD:2 * HD + 4],
            w1[2 * HD + 4:2 * HD + 5])


def _pad_col(w, b, width=8):
    wp = jnp.pad(w, ((0, 0), (0, width - w.shape[1])))
    bp = jnp.pad(b.reshape(1, -1), ((0, 0), (0, width - b.shape[0])))
    return wp, bp


def kernel(node_feat, node_pos, node_vel, edge_index, edge_attr, batch,
           params):
    row = edge_index[0]
    col = edge_index[1]
    pos4 = jnp.concatenate([node_pos, jnp.zeros((N, 1), jnp.float32)], axis=1)
    vel4 = jnp.concatenate([node_vel, jnp.zeros((N, 1), jnp.float32)], axis=1)

    w1a0, w1b0, _, _ = _split_w1(params["layers"][0])
    cp = params["color"]
    feat, a, b = _prologue_call(
        node_feat, pos4, batch.reshape(1, N), batch.reshape(N, 1),
        params["emb"]["W"], params["emb"]["b"].reshape(1, HD),
        cp["W1"], cp["b1"].reshape(1, HD), cp["W2"], cp["b2"].reshape(1, HD),
        w1a0, w1b0,
    )

    zs = jnp.zeros((N, SW), jnp.float32)
    zw = jnp.zeros((HD, HD), jnp.float32)
    _sc_gather, _sc_scatter = _sc_kernels()

    for l in range(NL):
        lp = params["layers"][l]
        _, _, w1c, w1d = _split_w1(lp)
        g, d4 = _sc_gather(row, col, a, b, pos4.reshape(N * 4))
        wp2, bp2 = _pad_col(lp["pos"]["W2"], lp["pos"]["b2"])
        eo = _edge_call(
            g, d4.reshape(E, 4), edge_attr,
            w1c, w1d, lp["msg"]["b1"].reshape(1, HD),
            lp["msg"]["W2"], lp["msg"]["b2"].reshape(1, HD),
            lp["pos"]["W1"], lp["pos"]["b1"].reshape(1, HD), wp2, bp2,
        )
        s = _sc_scatter(row, eo, zs)
        if l + 1 < NL:
            wa, wb, _, _ = _split_w1(params["layers"][l + 1])
        else:
            wa, wb = zw, zw
        wv2, bv2 = _pad_col(lp["vel"]["W2"], lp["vel"]["b2"])
        nd = lp["node"]
        feat, a, b, pos4 = _node_call(
            feat, s, pos4, vel4,
            nd["W1"][:HD], nd["W1"][HD:], nd["b1"].reshape(1, HD),
            nd["W2"], nd["b2"].reshape(1, HD),
            lp["vel"]["W1"], lp["vel"]["b1"].reshape(1, HD), wv2, bv2,
            wa, wb,
        )

    return (feat, pos4[:, :3])
